# baseline (speedup 1.0000x reference)
import functools

import jax
import jax.numpy as jnp
from jax import lax
from jax.experimental import pallas as pl
from jax.experimental.pallas import tpu as pltpu
from jax.experimental.pallas import tpu_sc as plsc

N_NODES = 10000
N_PAD = 10240
D = 128
NC = 2
NS = 16
CHUNK = 128
LANES = 16
ROWS_PER_TILE = N_PAD // NS


def _lane_broadcast(vec, lane):
    idx = jnp.full((LANES, 1), lane, dtype=jnp.int32)
    dnums = lax.GatherDimensionNumbers(
        offset_dims=(), collapsed_slice_dims=(0,), start_index_map=(0,))
    return lax.gather(vec, idx, dnums, slice_sizes=(1,),
                      mode=lax.GatherScatterMode.PROMISE_IN_BOUNDS)


def _make_seg_sum(n_chunks_per_tile):
    mesh = plsc.VectorSubcoreMesh(core_axis_name="c", subcore_axis_name="s")

    @functools.partial(
        pl.kernel,
        mesh=mesh,
        out_type=jax.ShapeDtypeStruct((NC, N_PAD, D), jnp.float32),
        scratch_types=[
            pltpu.VMEM((n_chunks_per_tile, CHUNK), jnp.int32),
            pltpu.VMEM((n_chunks_per_tile, CHUNK), jnp.int32),
            pltpu.VMEM((n_chunks_per_tile, CHUNK), jnp.float32),
            pltpu.VMEM((CHUNK, D), jnp.float32),
            pltpu.VMEM_SHARED((N_PAD, D), jnp.float32),
            pltpu.SemaphoreType.DMA,
        ],
    )
    def seg_sum(x_hbm, src_hbm, dst_hbm, w_hbm, out_hbm,
                src_v, dst_v, w_v, rows_v, acc, sem):
        c = lax.axis_index("c")
        s = lax.axis_index("s")
        tile = c * NS + s

        def _zero_row(i, carry):
            for j in range(D // LANES):
                rows_v[i, pl.ds(j * LANES, LANES)] = jnp.zeros((LANES,), jnp.float32)
            return carry
        lax.fori_loop(0, CHUNK, _zero_row, 0)

        base = s * ROWS_PER_TILE
        for t in range(ROWS_PER_TILE // CHUNK):
            pltpu.sync_copy(rows_v, acc.at[pl.ds(base + t * CHUNK, CHUNK)])

        ebase = tile * n_chunks_per_tile
        pltpu.sync_copy(src_hbm.at[pl.ds(ebase, n_chunks_per_tile)], src_v)
        pltpu.sync_copy(dst_hbm.at[pl.ds(ebase, n_chunks_per_tile)], dst_v)
        pltpu.sync_copy(w_hbm.at[pl.ds(ebase, n_chunks_per_tile)], w_v)

        plsc.subcore_barrier()

        def _chunk(ci, carry):
            pltpu.async_copy(x_hbm.at[src_v.at[ci]], rows_v, sem).wait()

            def _group(g, inner):
                w16 = w_v[ci, pl.ds(g * LANES, LANES)]
                for l in range(LANES):
                    wb = _lane_broadcast(w16, l)
                    e = g * LANES + l
                    for j in range(D // LANES):
                        rows_v[e, pl.ds(j * LANES, LANES)] = (
                            rows_v[e, pl.ds(j * LANES, LANES)] * wb)
                return inner
            lax.fori_loop(0, CHUNK // LANES, _group, 0)

            pltpu.sync_copy(rows_v, acc.at[dst_v.at[ci]], add=True)
            return carry
        lax.fori_loop(0, n_chunks_per_tile, _chunk, 0)

        plsc.subcore_barrier()

        pltpu.sync_copy(acc.at[pl.ds(base, ROWS_PER_TILE)],
                        out_hbm.at[c, pl.ds(base, ROWS_PER_TILE)])

    return seg_sum


def _dense(partials, x, w_rel, b_rel, w_root, relu):
    n_rows = x.shape[0]
    nb = 10
    br = n_rows // nb

    def body(p0_r, p1_r, x_r, wrel_r, b_r, wroot_r, o_r):
        agg = p0_r[...] + p1_r[...]
        acc = jnp.dot(agg, wrel_r[...], preferred_element_type=jnp.float32)
        acc = acc + jnp.dot(x_r[...], wroot_r[...],
                            preferred_element_type=jnp.float32)
        acc = acc + b_r[...]
        if relu:
            acc = jnp.maximum(acc, 0.0)
        o_r[...] = acc

    return pl.pallas_call(
        body,
        grid=(nb,),
        in_specs=[
            pl.BlockSpec((br, D), lambda i: (i, 0)),
            pl.BlockSpec((br, D), lambda i: (i, 0)),
            pl.BlockSpec((br, D), lambda i: (i, 0)),
            pl.BlockSpec((D, D), lambda i: (0, 0)),
            pl.BlockSpec((1, D), lambda i: (0, 0)),
            pl.BlockSpec((D, D), lambda i: (0, 0)),
        ],
        out_specs=pl.BlockSpec((br, D), lambda i: (i, 0)),
        out_shape=jax.ShapeDtypeStruct((n_rows, D), jnp.float32),
    )(partials[0], partials[1], x, w_rel, b_rel.reshape(1, D), w_root)


def kernel(x, edge_index, edge_weight, W_rel1, b_rel1, W_root1,
           W_rel2, b_rel2, W_root2):
    n_edges = edge_weight.shape[0]
    src = edge_index[0].astype(jnp.int32)
    dst = edge_index[1].astype(jnp.int32)
    w = edge_weight.astype(jnp.float32)

    per_round = NC * NS * CHUNK
    n_chunks_per_tile = -(-n_edges // per_round)
    n_chunks_per_tile += (-n_chunks_per_tile) % 8
    pad = n_chunks_per_tile * per_round - n_edges
    if pad:
        pad_dst = (jnp.arange(pad, dtype=jnp.int32) % N_PAD)
        src = jnp.concatenate([src, jnp.zeros((pad,), jnp.int32)])
        dst = jnp.concatenate([dst, pad_dst])
        w = jnp.concatenate([w, jnp.zeros((pad,), jnp.float32)])

    src2 = src.reshape(NC * NS * n_chunks_per_tile, CHUNK)
    dst2 = dst.reshape(NC * NS * n_chunks_per_tile, CHUNK)
    w2 = w.reshape(NC * NS * n_chunks_per_tile, CHUNK)

    seg_sum = _make_seg_sum(n_chunks_per_tile)

    p1 = seg_sum(x, src2, dst2, w2)[:, :N_NODES]
    h = _dense(p1, x, W_rel1, b_rel1, W_root1, relu=True)
    p2 = seg_sum(h, src2, dst2, w2)[:, :N_NODES]
    out = _dense(p2, h, W_rel2, b_rel2, W_root2, relu=False)
    return out


# speedup vs baseline: 1.1043x; 1.1043x over previous
import functools

import jax
import jax.numpy as jnp
from jax import lax
from jax.experimental import pallas as pl
from jax.experimental.pallas import tpu as pltpu
from jax.experimental.pallas import tpu_sc as plsc

N_NODES = 10000
N_PAD = 10240
D = 128
NC = 2
NS = 16
CHUNK = 128
LANES = 16
G = 8
ROWS_PER_TILE = N_PAD // NS


def _lane_broadcast(vec, lane):
    idx = jnp.full((LANES, 1), lane, dtype=jnp.int32)
    dnums = lax.GatherDimensionNumbers(
        offset_dims=(), collapsed_slice_dims=(0,), start_index_map=(0,))
    return lax.gather(vec, idx, dnums, slice_sizes=(1,),
                      mode=lax.GatherScatterMode.PROMISE_IN_BOUNDS)


def _make_seg_sum(nc):
    mesh = plsc.VectorSubcoreMesh(core_axis_name="c", subcore_axis_name="s")
    ngroups = nc // G

    @functools.partial(
        pl.kernel,
        mesh=mesh,
        out_type=jax.ShapeDtypeStruct((NC, N_PAD, D), jnp.float32),
        scratch_types=[
            pltpu.VMEM((2, G, CHUNK), jnp.int32),
            pltpu.VMEM((2, G, CHUNK), jnp.int32),
            pltpu.VMEM((2, G, CHUNK), jnp.float32),
            pltpu.VMEM((CHUNK, D), jnp.float32),
            pltpu.VMEM((CHUNK, D), jnp.float32),
            pltpu.VMEM_SHARED((N_PAD, D), jnp.float32),
            pltpu.SemaphoreType.DMA,
            pltpu.SemaphoreType.DMA,
            pltpu.SemaphoreType.DMA,
            pltpu.SemaphoreType.DMA,
            pltpu.SemaphoreType.DMA,
        ],
    )
    def seg_sum(x_hbm, src_hbm, dst_hbm, w_hbm, out_hbm,
                sg, dg, wg, b0, b1, acc,
                stg_sem, gsem0, gsem1, ssem0, ssem1):
        c = lax.axis_index("c")
        s = lax.axis_index("s")
        tile = c * NS + s
        buf = (b0, b1)
        gsem, ssem = (gsem0, gsem1), (ssem0, ssem1)

        def stage_start(gi):
            slot = lax.rem(gi, 2)
            eb = tile * nc + gi * G
            pltpu.async_copy(src_hbm.at[pl.ds(eb, G)], sg.at[slot], stg_sem)
            pltpu.async_copy(dst_hbm.at[pl.ds(eb, G)], dg.at[slot], stg_sem)
            pltpu.async_copy(w_hbm.at[pl.ds(eb, G)], wg.at[slot], stg_sem)

        def stage_wait():
            pltpu.make_async_copy(src_hbm.at[pl.ds(0, G)], sg.at[0], stg_sem).wait()
            pltpu.make_async_copy(dst_hbm.at[pl.ds(0, G)], dg.at[0], stg_sem).wait()
            pltpu.make_async_copy(w_hbm.at[pl.ds(0, G)], wg.at[0], stg_sem).wait()

        def start_g(k, b):
            slot, row = lax.rem(k // G, 2), lax.rem(k, G)
            pltpu.async_copy(x_hbm.at[sg.at[slot, row]], buf[b], gsem[b])

        def wait_g(b):
            pltpu.make_async_copy(x_hbm.at[sg.at[0, 0]], buf[b], gsem[b]).wait()

        def start_s(k, b):
            slot, row = lax.rem(k // G, 2), lax.rem(k, G)
            pltpu.async_copy(buf[b], acc.at[dg.at[slot, row]], ssem[b], add=True)

        def wait_s(b):
            pltpu.make_async_copy(buf[b], acc.at[dg.at[0, 0]], ssem[b]).wait()

        def scale(k, b):
            slot, row = lax.rem(k // G, 2), lax.rem(k, G)

            def _group(g, carry):
                w16 = wg[slot, row, pl.ds(g * LANES, LANES)]
                for l in range(LANES):
                    wb = _lane_broadcast(w16, l)
                    e = g * LANES + l
                    for j in range(D // LANES):
                        buf[b][e, pl.ds(j * LANES, LANES)] = (
                            buf[b][e, pl.ds(j * LANES, LANES)] * wb)
                return carry
            lax.fori_loop(0, CHUNK // LANES, _group, 0)

        def _zero_row(i, carry):
            for j in range(D // LANES):
                b0[i, pl.ds(j * LANES, LANES)] = jnp.zeros((LANES,), jnp.float32)
            return carry
        lax.fori_loop(0, CHUNK, _zero_row, 0)

        base = s * ROWS_PER_TILE
        for t in range(ROWS_PER_TILE // CHUNK):
            pltpu.sync_copy(b0, acc.at[pl.ds(base + t * CHUNK, CHUNK)])

        plsc.subcore_barrier()

        stage_start(0)
        stage_wait()
        stage_start(1)
        start_g(0, 0)
        start_g(1, 1)

        def _round(r, carry):
            for b in range(2):
                k = 2 * r + b
                wait_g(b)
                scale(k, b)
                start_s(k, b)
            for b in range(2):
                j = 2 * r + 2 + b

                @pl.when(j < nc)
                def _():
                    @pl.when(jnp.logical_and(lax.rem(j, G) == 0,
                                             j // G + 1 < ngroups))
                    def _():
                        stage_start(j // G + 1)

                    @pl.when(lax.rem(j, G) == 0)
                    def _():
                        stage_wait()

                    wait_s(b)
                    start_g(j, b)
            return carry
        lax.fori_loop(0, nc // 2, _round, 0)

        for b in range(2):
            wait_s(b)

        plsc.subcore_barrier()

        pltpu.sync_copy(acc.at[pl.ds(base, ROWS_PER_TILE)],
                        out_hbm.at[c, pl.ds(base, ROWS_PER_TILE)])

    return seg_sum


def _dense(partials, x, w_rel, b_rel, w_root, relu):
    n_rows = x.shape[0]
    nb = 10
    br = n_rows // nb

    def body(p_r, x_r, wrel_r, b_r, wroot_r, o_r):
        agg = p_r[0] + p_r[1]
        acc = jnp.dot(agg, wrel_r[...], preferred_element_type=jnp.float32)
        acc = acc + jnp.dot(x_r[...], wroot_r[...],
                            preferred_element_type=jnp.float32)
        acc = acc + b_r[...]
        if relu:
            acc = jnp.maximum(acc, 0.0)
        o_r[...] = acc

    return pl.pallas_call(
        body,
        grid=(nb,),
        in_specs=[
            pl.BlockSpec((NC, br, D), lambda i: (0, i, 0)),
            pl.BlockSpec((br, D), lambda i: (i, 0)),
            pl.BlockSpec((D, D), lambda i: (0, 0)),
            pl.BlockSpec((1, D), lambda i: (0, 0)),
            pl.BlockSpec((D, D), lambda i: (0, 0)),
        ],
        out_specs=pl.BlockSpec((br, D), lambda i: (i, 0)),
        out_shape=jax.ShapeDtypeStruct((n_rows, D), jnp.float32),
    )(partials, x, w_rel, b_rel.reshape(1, D), w_root)


def kernel(x, edge_index, edge_weight, W_rel1, b_rel1, W_root1,
           W_rel2, b_rel2, W_root2):
    n_edges = edge_weight.shape[0]
    src = edge_index[0].astype(jnp.int32)
    dst = edge_index[1].astype(jnp.int32)
    w = edge_weight.astype(jnp.float32)

    per_round = NC * NS * CHUNK
    nc = -(-n_edges // per_round)
    nc += (-nc) % G
    pad = nc * per_round - n_edges
    if pad:
        pad_dst = jnp.arange(pad, dtype=jnp.int32) % N_PAD
        src = jnp.concatenate([src, jnp.zeros((pad,), jnp.int32)])
        dst = jnp.concatenate([dst, pad_dst])
        w = jnp.concatenate([w, jnp.zeros((pad,), jnp.float32)])

    src2 = src.reshape(NC * NS * nc, CHUNK)
    dst2 = dst.reshape(NC * NS * nc, CHUNK)
    w2 = w.reshape(NC * NS * nc, CHUNK)

    seg_sum = _make_seg_sum(nc)

    p = seg_sum(x, src2, dst2, w2)[:, :N_NODES]
    h = _dense(p, x, W_rel1, b_rel1, W_root1, relu=True)
    q = seg_sum(h, src2, dst2, w2)[:, :N_NODES]
    out = _dense(q, h, W_rel2, b_rel2, W_root2, relu=False)
    return out


# speedup vs baseline: 1.2367x; 1.1198x over previous
import functools

import jax
import jax.numpy as jnp
from jax import lax
from jax.experimental import pallas as pl
from jax.experimental.pallas import tpu as pltpu
from jax.experimental.pallas import tpu_sc as plsc

N_NODES = 10000
N_PAD = 10240
D = 128
NC = 2
NS = 16
CHUNK = 128
LANES = 16
G = 8
ROWS_PER_TILE = N_PAD // NS


def _lane_broadcast(vec, lane):
    idx = jnp.full((LANES, 1), lane, dtype=jnp.int32)
    dnums = lax.GatherDimensionNumbers(
        offset_dims=(), collapsed_slice_dims=(0,), start_index_map=(0,))
    return lax.gather(vec, idx, dnums, slice_sizes=(1,),
                      mode=lax.GatherScatterMode.PROMISE_IN_BOUNDS)


def _make_seg_sum(nc):
    mesh = plsc.VectorSubcoreMesh(core_axis_name="c", subcore_axis_name="s")
    ngroups = nc // G

    @functools.partial(
        pl.kernel,
        mesh=mesh,
        out_type=jax.ShapeDtypeStruct((NC, N_PAD, D), jnp.float32),
        scratch_types=[
            pltpu.VMEM((2, G, CHUNK), jnp.int32),
            pltpu.VMEM((2, G, CHUNK), jnp.int32),
            pltpu.VMEM((2, G, CHUNK), jnp.float32),
            pltpu.VMEM((CHUNK, D), jnp.float32),
            pltpu.VMEM((CHUNK, D), jnp.float32),
            pltpu.VMEM_SHARED((N_PAD, D), jnp.float32),
            pltpu.SemaphoreType.DMA,
            pltpu.SemaphoreType.DMA,
            pltpu.SemaphoreType.DMA,
            pltpu.SemaphoreType.DMA,
            pltpu.SemaphoreType.DMA,
        ],
    )
    def seg_sum(x_hbm, src_hbm, dst_hbm, w_hbm, out_hbm,
                sg, dg, wg, b0, b1, acc,
                stg_sem, gsem0, gsem1, ssem0, ssem1):
        c = lax.axis_index("c")
        s = lax.axis_index("s")
        tile = c * NS + s
        buf = (b0, b1)
        gsem, ssem = (gsem0, gsem1), (ssem0, ssem1)

        def stage_start(gi):
            slot = lax.rem(gi, 2)
            eb = tile * nc + gi * G
            pltpu.async_copy(src_hbm.at[pl.ds(eb, G)], sg.at[slot], stg_sem)
            pltpu.async_copy(dst_hbm.at[pl.ds(eb, G)], dg.at[slot], stg_sem)
            pltpu.async_copy(w_hbm.at[pl.ds(eb, G)], wg.at[slot], stg_sem)

        def stage_wait():
            pltpu.make_async_copy(src_hbm.at[pl.ds(0, G)], sg.at[0], stg_sem).wait()
            pltpu.make_async_copy(dst_hbm.at[pl.ds(0, G)], dg.at[0], stg_sem).wait()
            pltpu.make_async_copy(w_hbm.at[pl.ds(0, G)], wg.at[0], stg_sem).wait()

        def start_g(k, b):
            slot, row = lax.rem(k // G, 2), lax.rem(k, G)
            pltpu.async_copy(x_hbm.at[sg.at[slot, row]], buf[b], gsem[b])

        def wait_g(b):
            pltpu.make_async_copy(x_hbm.at[sg.at[0, 0]], buf[b], gsem[b]).wait()

        def start_s(k, b):
            slot, row = lax.rem(k // G, 2), lax.rem(k, G)
            pltpu.async_copy(buf[b], acc.at[dg.at[slot, row]], ssem[b], add=True)

        def wait_s(b):
            pltpu.make_async_copy(buf[b], acc.at[dg.at[0, 0]], ssem[b]).wait()

        def scale(k, b):
            slot, row = lax.rem(k // G, 2), lax.rem(k, G)

            def _group(g, carry):
                w16 = wg[slot, row, pl.ds(g * LANES, LANES)]
                for l in range(LANES):
                    wb = _lane_broadcast(w16, l)
                    e = g * LANES + l
                    for j in range(D // LANES):
                        buf[b][e, pl.ds(j * LANES, LANES)] = (
                            buf[b][e, pl.ds(j * LANES, LANES)] * wb)
                return carry
            lax.fori_loop(0, CHUNK // LANES, _group, 0)

        def _zero_row(i, carry):
            for j in range(D // LANES):
                b0[i, pl.ds(j * LANES, LANES)] = jnp.zeros((LANES,), jnp.float32)
            return carry
        lax.fori_loop(0, CHUNK, _zero_row, 0)

        base = s * ROWS_PER_TILE
        for t in range(ROWS_PER_TILE // CHUNK):
            pltpu.sync_copy(b0, acc.at[pl.ds(base + t * CHUNK, CHUNK)])

        plsc.subcore_barrier()

        stage_start(0)
        stage_wait()
        stage_start(1)
        start_g(0, 0)
        start_g(1, 1)

        def _round(r, carry):
            for b in range(2):
                k = 2 * r + b
                wait_g(b)
                scale(k, b)
                start_s(k, b)
            for b in range(2):
                j = 2 * r + 2 + b

                @pl.when(j < nc)
                def _():
                    @pl.when(jnp.logical_and(lax.rem(j, G) == 0,
                                             j // G + 1 < ngroups))
                    def _():
                        stage_start(j // G + 1)

                    @pl.when(lax.rem(j, G) == 0)
                    def _():
                        stage_wait()

                    wait_s(b)
                    start_g(j, b)
            return carry
        lax.fori_loop(0, nc // 2, _round, 0)

        for b in range(2):
            wait_s(b)

        plsc.subcore_barrier()

        pltpu.sync_copy(acc.at[pl.ds(base, ROWS_PER_TILE)],
                        out_hbm.at[c, pl.ds(base, ROWS_PER_TILE)])

    return seg_sum


def _dense(partials, x, w_rel, b_rel, w_root, relu):
    n_rows = x.shape[0]
    nb = 10
    br = n_rows // nb

    def body(p_r, x_r, wrel_r, b_r, wroot_r, o_r):
        agg = p_r[0] + p_r[1]
        acc = jnp.dot(agg, wrel_r[...], preferred_element_type=jnp.float32)
        acc = acc + jnp.dot(x_r[...], wroot_r[...],
                            preferred_element_type=jnp.float32)
        acc = acc + b_r[...]
        if relu:
            acc = jnp.maximum(acc, 0.0)
        o_r[...] = acc

    return pl.pallas_call(
        body,
        grid=(nb,),
        in_specs=[
            pl.BlockSpec((NC, br, D), lambda i: (0, i, 0)),
            pl.BlockSpec((br, D), lambda i: (i, 0)),
            pl.BlockSpec((D, D), lambda i: (0, 0)),
            pl.BlockSpec((1, D), lambda i: (0, 0)),
            pl.BlockSpec((D, D), lambda i: (0, 0)),
        ],
        out_specs=pl.BlockSpec((br, D), lambda i: (i, 0)),
        out_shape=jax.ShapeDtypeStruct((n_rows, D), jnp.float32),
    )(partials, x, w_rel, b_rel.reshape(1, D), w_root)


def kernel(x, edge_index, edge_weight, W_rel1, b_rel1, W_root1,
           W_rel2, b_rel2, W_root2):
    n_edges = edge_weight.shape[0]
    src = edge_index[0].astype(jnp.int32)
    dst = edge_index[1].astype(jnp.int32)
    w = edge_weight.astype(jnp.float32)

    per_round = NC * NS * CHUNK
    nc = -(-n_edges // per_round)
    nc += (-nc) % G
    pad = nc * per_round - n_edges
    if pad:
        pad_dst = jnp.arange(pad, dtype=jnp.int32) % N_PAD
        src = jnp.concatenate([src, jnp.zeros((pad,), jnp.int32)])
        dst = jnp.concatenate([dst, pad_dst])
        w = jnp.concatenate([w, jnp.zeros((pad,), jnp.float32)])

    half = NS * nc * CHUNK
    src = src.at[half:].add(x.shape[0])
    src2 = src.reshape(NC * NS * nc, CHUNK)
    dst2 = dst.reshape(NC * NS * nc, CHUNK)
    w2 = w.reshape(NC * NS * nc, CHUNK)

    seg_sum = _make_seg_sum(nc)

    x_both = jnp.concatenate([x, x])
    p = seg_sum(x_both, src2, dst2, w2)[:, :N_NODES]
    h = _dense(p, x, W_rel1, b_rel1, W_root1, relu=True)
    h_both = jnp.concatenate([h, h])
    q = seg_sum(h_both, src2, dst2, w2)[:, :N_NODES]
    out = _dense(q, h, W_rel2, b_rel2, W_root2, relu=False)
    return out


# speedup vs baseline: 1.3513x; 1.0926x over previous
import functools

import jax
import jax.numpy as jnp
from jax import lax
from jax.experimental import pallas as pl
from jax.experimental.pallas import tpu as pltpu
from jax.experimental.pallas import tpu_sc as plsc

N_NODES = 10000
N_PAD = 10240
D = 128
NC = 2
NS = 16
CHUNK = 128
LANES = 16
G = 8
ROWS_PER_TILE = N_PAD // NS


def _lane_broadcast(vec, lane):
    idx = jnp.full((LANES, 1), lane, dtype=jnp.int32)
    dnums = lax.GatherDimensionNumbers(
        offset_dims=(), collapsed_slice_dims=(0,), start_index_map=(0,))
    return lax.gather(vec, idx, dnums, slice_sizes=(1,),
                      mode=lax.GatherScatterMode.PROMISE_IN_BOUNDS)


def _make_seg_sum(m0, m1):
    mesh = plsc.VectorSubcoreMesh(core_axis_name="c", subcore_axis_name="s")

    @functools.partial(
        pl.kernel,
        mesh=mesh,
        out_type=jax.ShapeDtypeStruct((NC, N_PAD, D), jnp.float32),
        scratch_types=[
            pltpu.VMEM((2, G, CHUNK), jnp.int32),
            pltpu.VMEM((2, G, CHUNK), jnp.int32),
            pltpu.VMEM((2, G, CHUNK), jnp.float32),
            pltpu.VMEM((CHUNK, D), jnp.float32),
            pltpu.VMEM((CHUNK, D), jnp.float32),
            pltpu.VMEM_SHARED((N_PAD, D), jnp.float32),
            pltpu.SemaphoreType.DMA,
            pltpu.SemaphoreType.DMA,
            pltpu.SemaphoreType.DMA,
            pltpu.SemaphoreType.DMA,
            pltpu.SemaphoreType.DMA,
        ],
    )
    def seg_sum(x_hbm, src_hbm, dst_hbm, w_hbm, out_hbm,
                sg, dg, wg, b0, b1, acc,
                stg_sem, gsem0, gsem1, ssem0, ssem1):
        c = lax.axis_index("c")
        s = lax.axis_index("s")
        buf = (b0, b1)
        gsem, ssem = (gsem0, gsem1), (ssem0, ssem1)
        m = jnp.where(c == 0, m0, m1)
        ngroups = m // G
        tbase = jnp.where(c == 0, s * m0, NS * m0 + s * m1)

        def stage_start(gi):
            slot = lax.rem(gi, 2)
            eb = tbase + gi * G
            pltpu.async_copy(src_hbm.at[pl.ds(eb, G)], sg.at[slot], stg_sem)
            pltpu.async_copy(dst_hbm.at[pl.ds(eb, G)], dg.at[slot], stg_sem)
            pltpu.async_copy(w_hbm.at[pl.ds(eb, G)], wg.at[slot], stg_sem)

        def stage_wait():
            pltpu.make_async_copy(src_hbm.at[pl.ds(0, G)], sg.at[0], stg_sem).wait()
            pltpu.make_async_copy(dst_hbm.at[pl.ds(0, G)], dg.at[0], stg_sem).wait()
            pltpu.make_async_copy(w_hbm.at[pl.ds(0, G)], wg.at[0], stg_sem).wait()

        def start_g(k, b):
            slot, row = lax.rem(k // G, 2), lax.rem(k, G)
            pltpu.async_copy(x_hbm.at[sg.at[slot, row]], buf[b], gsem[b])

        def wait_g(b):
            pltpu.make_async_copy(x_hbm.at[sg.at[0, 0]], buf[b], gsem[b]).wait()

        def start_s(k, b):
            slot, row = lax.rem(k // G, 2), lax.rem(k, G)
            pltpu.async_copy(buf[b], acc.at[dg.at[slot, row]], ssem[b], add=True)

        def wait_s(b):
            pltpu.make_async_copy(buf[b], acc.at[dg.at[0, 0]], ssem[b]).wait()

        def scale(k, b):
            slot, row = lax.rem(k // G, 2), lax.rem(k, G)

            def _group(g, carry):
                w16 = wg[slot, row, pl.ds(g * LANES, LANES)]
                for l in range(LANES):
                    wb = _lane_broadcast(w16, l)
                    e = g * LANES + l
                    for j in range(D // LANES):
                        buf[b][e, pl.ds(j * LANES, LANES)] = (
                            buf[b][e, pl.ds(j * LANES, LANES)] * wb)
                return carry
            lax.fori_loop(0, CHUNK // LANES, _group, 0)

        def _zero_row(i, carry):
            for j in range(D // LANES):
                b0[i, pl.ds(j * LANES, LANES)] = jnp.zeros((LANES,), jnp.float32)
            return carry
        lax.fori_loop(0, CHUNK, _zero_row, 0)

        base = s * ROWS_PER_TILE
        for t in range(ROWS_PER_TILE // CHUNK):
            pltpu.sync_copy(b0, acc.at[pl.ds(base + t * CHUNK, CHUNK)])

        plsc.subcore_barrier()

        stage_start(0)
        stage_wait()
        stage_start(1)
        start_g(0, 0)
        start_g(1, 1)

        def _round(r, carry):
            for b in range(2):
                k = 2 * r + b
                wait_g(b)
                scale(k, b)
                start_s(k, b)
            for b in range(2):
                j = 2 * r + 2 + b

                @pl.when(j < m)
                def _():
                    @pl.when(jnp.logical_and(lax.rem(j, G) == 0,
                                             j // G + 1 < ngroups))
                    def _():
                        stage_start(j // G + 1)

                    @pl.when(lax.rem(j, G) == 0)
                    def _():
                        stage_wait()

                    wait_s(b)
                    start_g(j, b)
            return carry
        lax.fori_loop(0, m // 2, _round, 0)

        for b in range(2):
            wait_s(b)

        plsc.subcore_barrier()

        pltpu.sync_copy(acc.at[pl.ds(base, ROWS_PER_TILE)],
                        out_hbm.at[c, pl.ds(base, ROWS_PER_TILE)])

    return seg_sum


def _dense(partials, x, w_rel, b_rel, w_root, relu):
    n_rows = x.shape[0]
    nb = 10
    br = n_rows // nb

    def body(p_r, x_r, wrel_r, b_r, wroot_r, o_r):
        agg = p_r[0] + p_r[1]
        acc = jnp.dot(agg, wrel_r[...], preferred_element_type=jnp.float32)
        acc = acc + jnp.dot(x_r[...], wroot_r[...],
                            preferred_element_type=jnp.float32)
        acc = acc + b_r[...]
        if relu:
            acc = jnp.maximum(acc, 0.0)
        o_r[...] = acc

    return pl.pallas_call(
        body,
        grid=(nb,),
        in_specs=[
            pl.BlockSpec((NC, br, D), lambda i: (0, i, 0)),
            pl.BlockSpec((br, D), lambda i: (i, 0)),
            pl.BlockSpec((D, D), lambda i: (0, 0)),
            pl.BlockSpec((1, D), lambda i: (0, 0)),
            pl.BlockSpec((D, D), lambda i: (0, 0)),
        ],
        out_specs=pl.BlockSpec((br, D), lambda i: (i, 0)),
        out_shape=jax.ShapeDtypeStruct((n_rows, D), jnp.float32),
    )(partials, x, w_rel, b_rel.reshape(1, D), w_root)


def kernel(x, edge_index, edge_weight, W_rel1, b_rel1, W_root1,
           W_rel2, b_rel2, W_root2):
    n_edges = edge_weight.shape[0]
    src = edge_index[0].astype(jnp.int32)
    dst = edge_index[1].astype(jnp.int32)
    w = edge_weight.astype(jnp.float32)

    per_round = NC * NS * CHUNK
    nc = -(-n_edges // per_round)
    nc += (-nc) % G
    m0 = (2 * nc) * 3 // 4
    m0 -= m0 % G
    m1 = 2 * nc - m0
    pad = nc * per_round - n_edges
    if pad:
        pad_dst = jnp.arange(pad, dtype=jnp.int32) % N_PAD
        src = jnp.concatenate([src, jnp.zeros((pad,), jnp.int32)])
        dst = jnp.concatenate([dst, pad_dst])
        w = jnp.concatenate([w, jnp.zeros((pad,), jnp.float32)])

    half = NS * nc * CHUNK
    src = src.at[half:].add(x.shape[0])
    src2 = src.reshape(NC * NS * nc, CHUNK)
    dst2 = dst.reshape(NC * NS * nc, CHUNK)
    w2 = w.reshape(NC * NS * nc, CHUNK)

    seg_sum = _make_seg_sum(m0, m1)

    x_both = jnp.concatenate([x, x])
    p = seg_sum(x_both, src2, dst2, w2)[:, :N_NODES]
    h = _dense(p, x, W_rel1, b_rel1, W_root1, relu=True)
    h_both = jnp.concatenate([h, h])
    q = seg_sum(h_both, src2, dst2, w2)[:, :N_NODES]
    out = _dense(q, h, W_rel2, b_rel2, W_root2, relu=False)
    return out


# speedup vs baseline: 1.4212x; 1.0518x over previous
import functools

import jax
import jax.numpy as jnp
from jax import lax
from jax.experimental import pallas as pl
from jax.experimental.pallas import tpu as pltpu
from jax.experimental.pallas import tpu_sc as plsc

N_NODES = 10000
N_PAD = 10240
D = 128
NC = 2
NS = 16
SUB = 64
LANES = 16
G = 16
NBUF = 4
ROWS_PER_TILE = N_PAD // NS


def _lane_broadcast(vec, lane):
    idx = jnp.full((LANES, 1), lane, dtype=jnp.int32)
    dnums = lax.GatherDimensionNumbers(
        offset_dims=(), collapsed_slice_dims=(0,), start_index_map=(0,))
    return lax.gather(vec, idx, dnums, slice_sizes=(1,),
                      mode=lax.GatherScatterMode.PROMISE_IN_BOUNDS)


def _make_seg_sum(m0, m1):
    mesh = plsc.VectorSubcoreMesh(core_axis_name="c", subcore_axis_name="s")

    @functools.partial(
        pl.kernel,
        mesh=mesh,
        out_type=jax.ShapeDtypeStruct((NC, N_PAD, D), jnp.float32),
        scratch_types=[
            pltpu.VMEM((2, G, SUB), jnp.int32),
            pltpu.VMEM((2, G, SUB), jnp.int32),
            pltpu.VMEM((2, G, SUB), jnp.float32),
            pltpu.VMEM((SUB, D), jnp.float32),
            pltpu.VMEM((SUB, D), jnp.float32),
            pltpu.VMEM((SUB, D), jnp.float32),
            pltpu.VMEM((SUB, D), jnp.float32),
            pltpu.VMEM_SHARED((N_PAD, D), jnp.float32),
            pltpu.SemaphoreType.DMA,
            pltpu.SemaphoreType.DMA,
            pltpu.SemaphoreType.DMA,
            pltpu.SemaphoreType.DMA,
            pltpu.SemaphoreType.DMA,
            pltpu.SemaphoreType.DMA,
            pltpu.SemaphoreType.DMA,
            pltpu.SemaphoreType.DMA,
            pltpu.SemaphoreType.DMA,
        ],
    )
    def seg_sum(x_hbm, src_hbm, dst_hbm, w_hbm, out_hbm,
                sg, dg, wg, b0, b1, b2, b3, acc,
                stg_sem, gsem0, gsem1, gsem2, gsem3,
                ssem0, ssem1, ssem2, ssem3):
        c = lax.axis_index("c")
        s = lax.axis_index("s")
        buf = (b0, b1, b2, b3)
        gsem = (gsem0, gsem1, gsem2, gsem3)
        ssem = (ssem0, ssem1, ssem2, ssem3)
        m = jnp.where(c == 0, m0, m1)
        ngroups = m // G
        tbase = jnp.where(c == 0, s * m0, NS * m0 + s * m1)

        def stage_start(gi):
            slot = lax.rem(gi, 2)
            eb = tbase + gi * G
            pltpu.async_copy(src_hbm.at[pl.ds(eb, G)], sg.at[slot], stg_sem)
            pltpu.async_copy(dst_hbm.at[pl.ds(eb, G)], dg.at[slot], stg_sem)
            pltpu.async_copy(w_hbm.at[pl.ds(eb, G)], wg.at[slot], stg_sem)

        def stage_wait():
            pltpu.make_async_copy(src_hbm.at[pl.ds(0, G)], sg.at[0], stg_sem).wait()
            pltpu.make_async_copy(dst_hbm.at[pl.ds(0, G)], dg.at[0], stg_sem).wait()
            pltpu.make_async_copy(w_hbm.at[pl.ds(0, G)], wg.at[0], stg_sem).wait()

        def start_g(k, b):
            slot, row = lax.rem(k // G, 2), lax.rem(k, G)
            pltpu.async_copy(x_hbm.at[sg.at[slot, row]], buf[b], gsem[b])

        def wait_g(b):
            pltpu.make_async_copy(x_hbm.at[sg.at[0, 0]], buf[b], gsem[b]).wait()

        def start_s(k, b):
            slot, row = lax.rem(k // G, 2), lax.rem(k, G)
            pltpu.async_copy(buf[b], acc.at[dg.at[slot, row]], ssem[b], add=True)

        def wait_s(b):
            pltpu.make_async_copy(buf[b], acc.at[dg.at[0, 0]], ssem[b]).wait()

        def scale(k, b):
            slot, row = lax.rem(k // G, 2), lax.rem(k, G)

            def _group(g, carry):
                w16 = wg[slot, row, pl.ds(g * LANES, LANES)]
                for l in range(LANES):
                    wb = _lane_broadcast(w16, l)
                    e = g * LANES + l
                    for j in range(D // LANES):
                        buf[b][e, pl.ds(j * LANES, LANES)] = (
                            buf[b][e, pl.ds(j * LANES, LANES)] * wb)
                return carry
            lax.fori_loop(0, SUB // LANES, _group, 0)

        def _zero_row(i, carry):
            for j in range(D // LANES):
                b0[i, pl.ds(j * LANES, LANES)] = jnp.zeros((LANES,), jnp.float32)
            return carry
        lax.fori_loop(0, SUB, _zero_row, 0)

        base = s * ROWS_PER_TILE
        for t in range(ROWS_PER_TILE // SUB):
            pltpu.sync_copy(b0, acc.at[pl.ds(base + t * SUB, SUB)])

        plsc.subcore_barrier()

        stage_start(0)
        stage_wait()
        stage_start(1)
        start_g(0, 0)
        start_g(1, 1)

        def _round(r, carry):
            for b in range(NBUF):
                k = NBUF * r + b
                wait_g(b)
                scale(k, b)
                start_s(k, b)
                b2 = (b + 2) % NBUF
                j = k + 2

                @pl.when(j < m)
                def _():
                    @pl.when(jnp.logical_and(lax.rem(j, G) == 0,
                                             j // G + 1 < ngroups))
                    def _():
                        stage_start(j // G + 1)

                    @pl.when(lax.rem(j, G) == 0)
                    def _():
                        stage_wait()

                    @pl.when(j >= NBUF)
                    def _():
                        wait_s(b2)

                    start_g(j, b2)
            return carry
        lax.fori_loop(0, m // NBUF, _round, 0)

        for b in range(NBUF):
            wait_s(b)

        plsc.subcore_barrier()

        pltpu.sync_copy(acc.at[pl.ds(base, ROWS_PER_TILE)],
                        out_hbm.at[c, pl.ds(base, ROWS_PER_TILE)])

    return seg_sum


def _dense(partials, x, w_rel, b_rel, w_root, relu):
    n_rows = x.shape[0]
    nb = 10
    br = n_rows // nb

    def body(p_r, x_r, wrel_r, b_r, wroot_r, o_r):
        agg = p_r[0] + p_r[1]
        acc = jnp.dot(agg, wrel_r[...], preferred_element_type=jnp.float32)
        acc = acc + jnp.dot(x_r[...], wroot_r[...],
                            preferred_element_type=jnp.float32)
        acc = acc + b_r[...]
        if relu:
            acc = jnp.maximum(acc, 0.0)
        o_r[...] = acc

    return pl.pallas_call(
        body,
        grid=(nb,),
        in_specs=[
            pl.BlockSpec((NC, br, D), lambda i: (0, i, 0)),
            pl.BlockSpec((br, D), lambda i: (i, 0)),
            pl.BlockSpec((D, D), lambda i: (0, 0)),
            pl.BlockSpec((1, D), lambda i: (0, 0)),
            pl.BlockSpec((D, D), lambda i: (0, 0)),
        ],
        out_specs=pl.BlockSpec((br, D), lambda i: (i, 0)),
        out_shape=jax.ShapeDtypeStruct((n_rows, D), jnp.float32),
    )(partials, x, w_rel, b_rel.reshape(1, D), w_root)


def kernel(x, edge_index, edge_weight, W_rel1, b_rel1, W_root1,
           W_rel2, b_rel2, W_root2):
    n_edges = edge_weight.shape[0]
    src = edge_index[0].astype(jnp.int32)
    dst = edge_index[1].astype(jnp.int32)
    w = edge_weight.astype(jnp.float32)

    mt = -(-n_edges // (NS * SUB))
    mt += (-mt) % (2 * G)
    m0 = mt * 3 // 4
    m0 -= m0 % G
    m1 = mt - m0
    pad = mt * NS * SUB - n_edges
    if pad:
        pad_dst = jnp.arange(pad, dtype=jnp.int32) % N_PAD
        src = jnp.concatenate([src, jnp.zeros((pad,), jnp.int32)])
        dst = jnp.concatenate([dst, pad_dst])
        w = jnp.concatenate([w, jnp.zeros((pad,), jnp.float32)])

    half = NS * m0 * SUB
    src = src.at[half:].add(x.shape[0])
    src2 = src.reshape(-1, SUB)
    dst2 = dst.reshape(-1, SUB)
    w2 = w.reshape(-1, SUB)

    seg_sum = _make_seg_sum(m0, m1)

    x_both = jnp.concatenate([x, x])
    p = seg_sum(x_both, src2, dst2, w2)[:, :N_NODES]
    h = _dense(p, x, W_rel1, b_rel1, W_root1, relu=True)
    h_both = jnp.concatenate([h, h])
    q = seg_sum(h_both, src2, dst2, w2)[:, :N_NODES]
    out = _dense(q, h, W_rel2, b_rel2, W_root2, relu=False)
    return out
